# Initial kernel scaffold; baseline (speedup 1.0000x reference)
import jax
import jax.numpy as jnp
from jax.experimental import pallas as pl


def kernel(x, edge, W0, gamma0, beta0, anchor, sigma_p, W1, gamma1, beta1):
    raise NotImplementedError("write your pallas kernel here")


# speedup vs baseline: 3.9927x; 3.9927x over previous
import functools

import jax
import jax.numpy as jnp
from jax.experimental import pallas as pl

_NODE = 32
_KNN = 5
_DN = (((1,), (1,)), ((), ()))
_DT = (((0,), (0,)), ((), ()))


def _stage_a(xt_ref, eg_ref, w0t_ref, g0_ref, b0_ref, anc_ref, sp_ref,
             nodes_ref):
    B, N, C = xt_ref.shape
    w0t = w0t_ref[...]
    hs = []
    for b in range(B):
        x1 = jax.nn.sigmoid(eg_ref[b]) * xt_ref[b]
        hs.append(jnp.dot(x1, w0t, preferred_element_type=jnp.float32))
    cnt = float(B * N)
    s1 = sum(h.sum(axis=0, keepdims=True) for h in hs) / cnt
    s2 = sum((h * h).sum(axis=0, keepdims=True) for h in hs) / cnt
    var = s2 - s1 * s1
    inv = jax.lax.rsqrt(var + 1e-5)
    g0 = g0_ref[...] * inv
    b0 = b0_ref[...] - s1 * g0_ref[...] * inv

    sig = jax.nn.sigmoid(sp_ref[...])
    anc = anc_ref[...]
    inv_s2 = 1.0 / (sig * sig)
    a_is2 = anc * inv_s2
    ones_c = jnp.ones((1, anc.shape[1]), jnp.float32)
    const = jax.lax.dot_general(ones_c, anc * a_is2, _DN,
                                preferred_element_type=jnp.float32)
    for b in range(B):
        z = jnp.maximum(hs[b] * g0 + b0, 0.0)
        q = jax.lax.dot_general(z * z, inv_s2, _DN,
                                preferred_element_type=jnp.float32)
        lx = jax.lax.dot_general(z, a_is2, _DN,
                                 preferred_element_type=jnp.float32)
        logits = -0.5 * (q - 2.0 * lx + const)
        m = logits.max(axis=1, keepdims=True)
        e = jnp.exp(logits - m)
        sa = e / e.sum(axis=1, keepdims=True)
        ones_n = jnp.ones((N, 1), jnp.float32)
        den = jax.lax.dot_general(sa, ones_n, _DT,
                                  preferred_element_type=jnp.float32)
        t = jax.lax.dot_general(sa, z, _DT,
                                preferred_element_type=jnp.float32)
        nodes = (t - anc * den) / sig / (den + 1e-9)
        rn = jnp.sqrt((nodes * nodes).sum(axis=1, keepdims=True))
        nodes = nodes / jnp.maximum(rn, 1e-12)
        gn = jnp.sqrt((nodes * nodes).sum(keepdims=True))
        nodes_ref[b] = nodes / jnp.maximum(gn, 1e-12)


def _stage_b(xt_ref, f_ref, w1_ref, g1_ref, b1_ref, out_ref):
    B, N, C = xt_ref.shape
    w1 = w1_ref[...]
    w1a = w1[:, :C]
    wd = w1[:, C:] - w1a
    s1 = jnp.zeros((1, C), jnp.float32)
    s2 = jnp.zeros((1, C), jnp.float32)
    saved = []
    iota = jax.lax.broadcasted_iota(jnp.int32, (N, _NODE), 1)
    for b in range(B):
        xb = xt_ref[b]
        fb = f_ref[b]
        g = jax.lax.dot_general(fb, w1a, _DN,
                                preferred_element_type=jnp.float32)
        p = jax.lax.dot_general(xb, wd, _DN,
                                preferred_element_type=jnp.float32)
        xs = (xb * xb).sum(axis=1, keepdims=True)
        ones_c = jnp.ones((1, C), jnp.float32)
        fs = jax.lax.dot_general(ones_c, fb * fb, _DN,
                                 preferred_element_type=jnp.float32)
        xdf = jax.lax.dot_general(xb, fb, _DN,
                                  preferred_element_type=jnp.float32)
        d2 = xs - 2.0 * xdf + fs
        gs_list = []
        for _ in range(_KNN):
            mn = d2.min(axis=1, keepdims=True)
            cand = jnp.where(d2 == mn, iota, _NODE)
            fi = cand.min(axis=1, keepdims=True)
            onehot = (iota == fi).astype(jnp.float32)
            d2 = jnp.where(iota == fi, jnp.inf, d2)
            gs_list.append(jnp.dot(onehot, g, preferred_element_type=jnp.float32))
        r3 = jnp.concatenate(gs_list, axis=0).reshape(N, _KNN, C)
        gmax = r3.max(axis=1)
        sg = r3.sum(axis=1)
        sg2 = (r3 * r3).sum(axis=1)
        s1 = s1 + sg.sum(axis=0, keepdims=True) + _KNN * p.sum(axis=0, keepdims=True)
        s2 = s2 + (sg2.sum(axis=0, keepdims=True)
                   + 2.0 * (sg * p).sum(axis=0, keepdims=True)
                   + _KNN * (p * p).sum(axis=0, keepdims=True))
        saved.append((gmax, p))
    cnt = float(B * N * _KNN)
    mean = s1 / cnt
    var = s2 / cnt - mean * mean
    a1 = g1_ref[...] * jax.lax.rsqrt(var + 1e-5)
    b1 = b1_ref[...] - a1 * mean
    for b in range(B):
        gmax, p = saved[b]
        y = jnp.maximum(a1 * (gmax + p) + b1, 0.0)
        out_ref[b] = xt_ref[b] + y


@jax.jit
def kernel(x, edge, W0, gamma0, beta0, anchor, sigma_p, W1, gamma1, beta1):
    B, C, H, W = x.shape
    N = H * W
    xt = x.reshape(B, C, N).transpose(0, 2, 1)
    eg = edge.reshape(B, 1, N).transpose(0, 2, 1)
    nodes = pl.pallas_call(
        _stage_a,
        out_shape=jax.ShapeDtypeStruct((B, _NODE, C), jnp.float32),
    )(xt, eg, W0.T, gamma0[None], beta0[None], anchor, sigma_p)
    nodes_feat = nodes.reshape(B, C, _NODE).transpose(0, 2, 1)
    out_t = pl.pallas_call(
        _stage_b,
        out_shape=jax.ShapeDtypeStruct((B, N, C), jnp.float32),
    )(xt, nodes_feat, W1, gamma1[None], beta1[None])
    return out_t.transpose(0, 2, 1).reshape(B, C, H, W)
